# baseline (speedup 1.0000x reference)
import functools

import jax
import jax.numpy as jnp
import numpy as np
from jax.experimental import pallas as pl
from jax.experimental.pallas import tpu as pltpu

_THR = 0.75
_NOC = -1


def _patchify_kernel(img_ref, p_ref, o_ref, *, C, Hp, Wp, p):
    img = img_ref[0, :, 0]
    pieces = []
    for c in range(C):
        v = img[c].reshape(Hp, p, Wp, p)
        v = v.transpose(0, 2, 1, 3)
        pieces.append(v.reshape(Hp * Wp, p * p))
    chunk = jnp.concatenate(pieces, axis=1)
    o_ref[...] = jax.lax.dot_general(
        chunk, p_ref[...], (((1,), (0,)), ((), ())),
        preferred_element_type=jnp.float32)


def _nn_kernel(x_ref, c_ref, a_ref, o_ref, min_ref, arg_ref, *, nt, bn):
    j = pl.program_id(0)

    @pl.when(j == 0)
    def _init():
        min_ref[...] = jnp.full_like(min_ref, jnp.inf)
        arg_ref[...] = jnp.zeros_like(arg_ref)

    c = c_ref[...]
    c2 = jnp.sum(c * c, axis=1, keepdims=True)
    c2m = jnp.where(a_ref[...] > 0, c2, jnp.inf)

    s = jax.lax.dot_general(c, x_ref[...], (((1,), (1,)), ((), ())),
                            preferred_element_type=jnp.float32)
    v = s + c2m
    tmin = jnp.min(v, axis=0, keepdims=True)
    iota = jax.lax.broadcasted_iota(jnp.int32, v.shape, 0)
    targ = jnp.min(jnp.where(v == tmin, iota, bn), axis=0, keepdims=True) + j * bn
    better = tmin < min_ref[...]
    arg_ref[...] = jnp.where(better, targ, arg_ref[...])
    min_ref[...] = jnp.where(better, tmin, min_ref[...])

    @pl.when(j == nt - 1)
    def _fin():
        xs = x_ref[...]
        ones = jnp.ones((1, xs.shape[1]), jnp.float32)
        x2 = jax.lax.dot_general(
            ones, xs * xs, (((1,), (1,)), ((), ())),
            preferred_element_type=jnp.float32) * 0.25
        mind = min_ref[...] + x2
        o_ref[...] = jnp.where(mind <= _THR, arg_ref[...], _NOC).astype(jnp.int32)


def _perm_matrix(p, C):
    D = p * p * C
    P = np.zeros((D, D), np.float32)
    for c in range(C):
        for q in range(p * p):
            P[c * p * p + q, q * C + c] = -2.0
    return jnp.asarray(P)


def kernel(imgs, patch_size, codes, active):
    B, C, T, H, W = imgs.shape
    N, D = codes.shape
    p = int(np.sqrt(D // C))
    Hp, Wp = H // p, W // p
    M = B * T * Hp * Wp
    R = Hp * Wp

    img6 = imgs.reshape(B, C, T, Hp, p, Wp * p)
    P = _perm_matrix(p, C)

    xs = pl.pallas_call(
        functools.partial(_patchify_kernel, C=C, Hp=Hp, Wp=Wp, p=p),
        grid=(B * T,),
        in_specs=[
            pl.BlockSpec((1, C, 1, Hp, p, Wp * p),
                         lambda k: (k // T, 0, k % T, 0, 0, 0)),
            pl.BlockSpec((D, D), lambda k: (0, 0)),
        ],
        out_specs=pl.BlockSpec((R, D), lambda k: (k, 0)),
        out_shape=jax.ShapeDtypeStruct((M, D), jnp.float32),
    )(img6, P)

    BN = 512
    NT = N // BN
    amask = active.astype(jnp.float32).reshape(N, 1)

    out = pl.pallas_call(
        functools.partial(_nn_kernel, nt=NT, bn=BN),
        grid=(NT,),
        in_specs=[
            pl.BlockSpec((M, D), lambda j: (0, 0)),
            pl.BlockSpec((BN, D), lambda j: (j, 0)),
            pl.BlockSpec((BN, 1), lambda j: (j, 0)),
        ],
        out_specs=pl.BlockSpec((1, M), lambda j: (0, 0)),
        out_shape=jax.ShapeDtypeStruct((1, M), jnp.int32),
        scratch_shapes=[
            pltpu.VMEM((1, M), jnp.float32),
            pltpu.VMEM((1, M), jnp.int32),
        ],
    )(xs, codes, amask)
    return out.reshape(B, T, Hp, Wp)


# speedup vs baseline: 1.1702x; 1.1702x over previous
import functools

import jax
import jax.numpy as jnp
import numpy as np
from jax.experimental import pallas as pl
from jax.experimental.pallas import tpu as pltpu

_THR = 0.75
_NOC = -1


def _patchify_kernel(img_ref, p_ref, o_ref, *, C, Hp, Wp, p):
    img = img_ref[0, :, 0].astype(jnp.bfloat16)
    pieces = []
    for c in range(C):
        v = img[c].reshape(Hp, p, Wp, p)
        v = v.transpose(0, 2, 1, 3)
        pieces.append(v.reshape(Hp * Wp, p * p))
    chunk = jnp.concatenate(pieces, axis=1)
    o_ref[...] = jax.lax.dot_general(
        chunk, p_ref[...], (((1,), (0,)), ((), ())),
        preferred_element_type=jnp.float32).astype(jnp.bfloat16)


def _nn_kernel(x_ref, c_ref, a_ref, o_ref, min_ref, arg_ref, *, nt, bn):
    j = pl.program_id(0)

    @pl.when(j == 0)
    def _init():
        min_ref[...] = jnp.full_like(min_ref, jnp.inf)
        arg_ref[...] = jnp.zeros_like(arg_ref)

    c = c_ref[...]
    c32 = c.astype(jnp.float32)
    c2 = jnp.sum(c32 * c32, axis=1, keepdims=True)
    c2m = jnp.where(a_ref[...] > 0, c2, jnp.inf)

    s = jax.lax.dot_general(c, x_ref[...], (((1,), (1,)), ((), ())),
                            preferred_element_type=jnp.float32)
    v = s + c2m
    tmin = jnp.min(v, axis=0, keepdims=True)
    iota = jax.lax.broadcasted_iota(jnp.int32, v.shape, 0)
    targ = jnp.min(jnp.where(v == tmin, iota, bn), axis=0, keepdims=True) + j * bn
    better = tmin < min_ref[...]
    arg_ref[...] = jnp.where(better, targ, arg_ref[...])
    min_ref[...] = jnp.where(better, tmin, min_ref[...])

    @pl.when(j == nt - 1)
    def _fin():
        xs = x_ref[...]
        ones = jnp.ones((1, xs.shape[1]), jnp.bfloat16)
        x2 = jax.lax.dot_general(
            ones, xs * xs, (((1,), (1,)), ((), ())),
            preferred_element_type=jnp.float32) * 0.25
        mind = min_ref[...] + x2
        o_ref[...] = jnp.where(mind <= _THR, arg_ref[...], _NOC).astype(jnp.int32)


def _perm_matrix(p, C):
    D = p * p * C
    P = np.zeros((D, D), np.float32)
    for c in range(C):
        for q in range(p * p):
            P[c * p * p + q, q * C + c] = -2.0
    return jnp.asarray(P, dtype=jnp.bfloat16)


def kernel(imgs, patch_size, codes, active):
    B, C, T, H, W = imgs.shape
    N, D = codes.shape
    p = int(np.sqrt(D // C))
    Hp, Wp = H // p, W // p
    M = B * T * Hp * Wp
    R = Hp * Wp

    img6 = imgs.reshape(B, C, T, Hp, p, Wp * p)
    P = _perm_matrix(p, C)

    xs = pl.pallas_call(
        functools.partial(_patchify_kernel, C=C, Hp=Hp, Wp=Wp, p=p),
        grid=(B * T,),
        in_specs=[
            pl.BlockSpec((1, C, 1, Hp, p, Wp * p),
                         lambda k: (k // T, 0, k % T, 0, 0, 0)),
            pl.BlockSpec((D, D), lambda k: (0, 0)),
        ],
        out_specs=pl.BlockSpec((R, D), lambda k: (k, 0)),
        out_shape=jax.ShapeDtypeStruct((M, D), jnp.bfloat16),
    )(img6, P)

    BN = 512
    NT = N // BN
    amask = active.astype(jnp.float32).reshape(N, 1)
    codes_bf = codes.astype(jnp.bfloat16)

    out = pl.pallas_call(
        functools.partial(_nn_kernel, nt=NT, bn=BN),
        grid=(NT,),
        in_specs=[
            pl.BlockSpec((M, D), lambda j: (0, 0)),
            pl.BlockSpec((BN, D), lambda j: (j, 0)),
            pl.BlockSpec((BN, 1), lambda j: (j, 0)),
        ],
        out_specs=pl.BlockSpec((1, M), lambda j: (0, 0)),
        out_shape=jax.ShapeDtypeStruct((1, M), jnp.int32),
        scratch_shapes=[
            pltpu.VMEM((1, M), jnp.float32),
            pltpu.VMEM((1, M), jnp.int32),
        ],
    )(xs, codes_bf, amask)
    return out.reshape(B, T, Hp, Wp)


# speedup vs baseline: 1.2076x; 1.0320x over previous
import functools

import jax
import jax.numpy as jnp
import numpy as np
from jax.experimental import pallas as pl
from jax.experimental.pallas import tpu as pltpu

_THR = 0.75
_NOC = -1


def _patchify_kernel(img_ref, p_ref, o_ref, *, C, Hp, Wp, p):
    img = img_ref[0, :, 0].astype(jnp.bfloat16)
    pieces = []
    for c in range(C):
        v = img[c].reshape(Hp, p, Wp, p)
        v = v.transpose(0, 2, 1, 3)
        pieces.append(v.reshape(Hp * Wp, p * p))
    chunk = jnp.concatenate(pieces, axis=1)
    o_ref[...] = jax.lax.dot_general(
        chunk, p_ref[...], (((1,), (0,)), ((), ())),
        preferred_element_type=jnp.float32).astype(jnp.bfloat16)


def _nn_kernel(x_ref, c_ref, a_ref, o_ref, min_ref, arg_ref, *, nt, bn):
    j = pl.program_id(0)

    @pl.when(j == 0)
    def _init():
        min_ref[...] = jnp.full_like(min_ref, jnp.inf)
        arg_ref[...] = jnp.zeros_like(arg_ref)

    c = c_ref[...]
    c32 = c.astype(jnp.float32)
    c2 = jnp.sum(c32 * c32, axis=1, keepdims=True)
    c2m = jnp.where(a_ref[...] > 0, c2, jnp.inf)

    s = jax.lax.dot_general(c, x_ref[...], (((1,), (1,)), ((), ())),
                            preferred_element_type=jnp.float32)
    v = s + c2m
    tmin = jnp.min(v, axis=0, keepdims=True)
    iota = jax.lax.broadcasted_iota(jnp.int32, v.shape, 0)
    targ = jnp.min(jnp.where(v == tmin, iota, bn), axis=0, keepdims=True) + j * bn
    better = tmin < min_ref[...]
    arg_ref[...] = jnp.where(better, targ, arg_ref[...])
    min_ref[...] = jnp.where(better, tmin, min_ref[...])

    @pl.when(j == nt - 1)
    def _fin():
        xs = x_ref[...]
        ones = jnp.ones((1, xs.shape[1]), jnp.bfloat16)
        x2 = jax.lax.dot_general(
            ones, xs * xs, (((1,), (1,)), ((), ())),
            preferred_element_type=jnp.float32) * 0.25
        mind = min_ref[...] + x2
        o_ref[...] = jnp.where(mind <= _THR, arg_ref[...], _NOC).astype(jnp.int32)


def _perm_matrix(p, C):
    D = p * p * C
    P = np.zeros((D, D), np.float32)
    for c in range(C):
        for q in range(p * p):
            P[c * p * p + q, q * C + c] = -2.0
    return jnp.asarray(P, dtype=jnp.bfloat16)


def kernel(imgs, patch_size, codes, active):
    B, C, T, H, W = imgs.shape
    N, D = codes.shape
    p = int(np.sqrt(D // C))
    Hp, Wp = H // p, W // p
    M = B * T * Hp * Wp
    R = Hp * Wp

    img6 = imgs.reshape(B, C, T, Hp, p, Wp * p)
    P = _perm_matrix(p, C)

    xs = pl.pallas_call(
        functools.partial(_patchify_kernel, C=C, Hp=Hp, Wp=Wp, p=p),
        grid=(B * T,),
        in_specs=[
            pl.BlockSpec((1, C, 1, Hp, p, Wp * p),
                         lambda k: (k // T, 0, k % T, 0, 0, 0)),
            pl.BlockSpec((D, D), lambda k: (0, 0)),
        ],
        out_specs=pl.BlockSpec((R, D), lambda k: (k, 0)),
        out_shape=jax.ShapeDtypeStruct((M, D), jnp.bfloat16),
    )(img6, P)

    BN = 1024
    NT = N // BN
    amask = active.astype(jnp.float32).reshape(N, 1)
    codes_bf = codes.astype(jnp.bfloat16)

    out = pl.pallas_call(
        functools.partial(_nn_kernel, nt=NT, bn=BN),
        grid=(NT,),
        in_specs=[
            pl.BlockSpec((M, D), lambda j: (0, 0)),
            pl.BlockSpec((BN, D), lambda j: (j, 0)),
            pl.BlockSpec((BN, 1), lambda j: (j, 0)),
        ],
        out_specs=pl.BlockSpec((1, M), lambda j: (0, 0)),
        out_shape=jax.ShapeDtypeStruct((1, M), jnp.int32),
        scratch_shapes=[
            pltpu.VMEM((1, M), jnp.float32),
            pltpu.VMEM((1, M), jnp.int32),
        ],
    )(xs, codes_bf, amask)
    return out.reshape(B, T, Hp, Wp)


# speedup vs baseline: 1.2081x; 1.0004x over previous
import functools

import jax
import jax.numpy as jnp
import numpy as np
from jax.experimental import pallas as pl
from jax.experimental.pallas import tpu as pltpu

_THR = 0.75
_NOC = -1


def _patchify_kernel(img_ref, p_ref, o_ref, *, C, Hp, Wp, p):
    img = img_ref[0, :, 0].astype(jnp.bfloat16)
    pieces = []
    for c in range(C):
        v = img[c].reshape(Hp, p, Wp, p)
        v = v.transpose(0, 2, 1, 3)
        pieces.append(v.reshape(Hp * Wp, p * p))
    chunk = jnp.concatenate(pieces, axis=1)
    o_ref[...] = jax.lax.dot_general(
        chunk, p_ref[...], (((1,), (0,)), ((), ())),
        preferred_element_type=jnp.float32).astype(jnp.bfloat16)


def _nn_kernel(x_ref, c_ref, a_ref, o_ref, min_ref, arg_ref, *, nt, bn):
    j = pl.program_id(0)

    @pl.when(j == 0)
    def _init():
        min_ref[...] = jnp.full_like(min_ref, jnp.inf)
        arg_ref[...] = jnp.zeros_like(arg_ref)

    c = c_ref[...]
    c32 = c.astype(jnp.float32)
    c2 = jnp.sum(c32 * c32, axis=1, keepdims=True)
    c2m = jnp.where(a_ref[...] > 0, c2, jnp.inf)

    s = jax.lax.dot_general(c, x_ref[...], (((1,), (1,)), ((), ())),
                            preferred_element_type=jnp.float32)
    v = s + c2m
    tmin = jnp.min(v, axis=0, keepdims=True)
    iota = jax.lax.broadcasted_iota(jnp.int32, v.shape, 0)
    targ = jnp.min(jnp.where(v == tmin, iota, bn), axis=0, keepdims=True) + j * bn
    better = tmin < min_ref[...]
    arg_ref[...] = jnp.where(better, targ, arg_ref[...])
    min_ref[...] = jnp.where(better, tmin, min_ref[...])

    @pl.when(j == nt - 1)
    def _fin():
        xs = x_ref[...]
        ones = jnp.ones((1, xs.shape[1]), jnp.bfloat16)
        x2 = jax.lax.dot_general(
            ones, xs * xs, (((1,), (1,)), ((), ())),
            preferred_element_type=jnp.float32) * 0.25
        mind = min_ref[...] + x2
        o_ref[...] = jnp.where(mind <= _THR, arg_ref[...], _NOC).astype(jnp.int32)


def _perm_matrix(p, C):
    D = p * p * C
    P = np.zeros((D, D), np.float32)
    for c in range(C):
        for q in range(p * p):
            P[c * p * p + q, q * C + c] = -2.0
    return jnp.asarray(P, dtype=jnp.bfloat16)


def kernel(imgs, patch_size, codes, active):
    B, C, T, H, W = imgs.shape
    N, D = codes.shape
    p = int(np.sqrt(D // C))
    Hp, Wp = H // p, W // p
    M = B * T * Hp * Wp
    R = Hp * Wp

    img6 = imgs.reshape(B, C, T, Hp, p, Wp * p)
    P = _perm_matrix(p, C)

    xs = pl.pallas_call(
        functools.partial(_patchify_kernel, C=C, Hp=Hp, Wp=Wp, p=p),
        grid=(B * T,),
        in_specs=[
            pl.BlockSpec((1, C, 1, Hp, p, Wp * p),
                         lambda k: (k // T, 0, k % T, 0, 0, 0)),
            pl.BlockSpec((D, D), lambda k: (0, 0)),
        ],
        out_specs=pl.BlockSpec((R, D), lambda k: (k, 0)),
        out_shape=jax.ShapeDtypeStruct((M, D), jnp.bfloat16),
    )(img6, P)

    BN = 2048
    NT = N // BN
    amask = active.astype(jnp.float32).reshape(N, 1)
    codes_bf = codes.astype(jnp.bfloat16)

    out = pl.pallas_call(
        functools.partial(_nn_kernel, nt=NT, bn=BN),
        grid=(NT,),
        in_specs=[
            pl.BlockSpec((M, D), lambda j: (0, 0)),
            pl.BlockSpec((BN, D), lambda j: (j, 0)),
            pl.BlockSpec((BN, 1), lambda j: (j, 0)),
        ],
        out_specs=pl.BlockSpec((1, M), lambda j: (0, 0)),
        out_shape=jax.ShapeDtypeStruct((1, M), jnp.int32),
        scratch_shapes=[
            pltpu.VMEM((1, M), jnp.float32),
            pltpu.VMEM((1, M), jnp.int32),
        ],
    )(xs, codes_bf, amask)
    return out.reshape(B, T, Hp, Wp)


# speedup vs baseline: 1.3238x; 1.0958x over previous
import functools

import jax
import jax.numpy as jnp
import numpy as np
from jax.experimental import pallas as pl
from jax.experimental.pallas import tpu as pltpu

_THR = 0.75
_NOC = -1


def _patchify_kernel(img_ref, p_ref, o_ref, *, C, Hp, Wp, p):
    img = img_ref[0, :, 0].astype(jnp.bfloat16)
    pieces = []
    for c in range(C):
        v = img[c].reshape(Hp, p, Wp, p)
        v = v.transpose(0, 2, 1, 3)
        pieces.append(v.reshape(Hp * Wp, p * p))
    chunk = jnp.concatenate(pieces, axis=1)
    o_ref[...] = jax.lax.dot_general(
        chunk, p_ref[...], (((1,), (0,)), ((), ())),
        preferred_element_type=jnp.float32).astype(jnp.bfloat16)


def _nn_kernel(x_ref, c_ref, a_ref, o_ref, min_ref, arg_ref, *, nt, bn):
    j = pl.program_id(0)

    @pl.when(j == 0)
    def _init():
        min_ref[...] = jnp.full_like(min_ref, jnp.inf)
        arg_ref[...] = jnp.zeros_like(arg_ref)

    c = c_ref[...]
    c2 = jnp.sum(c * c, axis=1, keepdims=True)
    c2m = jnp.where(a_ref[...] > 0, c2, jnp.inf)

    s = jax.lax.dot_general(c.astype(jnp.bfloat16), x_ref[...],
                            (((1,), (1,)), ((), ())),
                            preferred_element_type=jnp.float32)
    v = s + c2m
    tmin = jnp.min(v, axis=0, keepdims=True)
    iota = jax.lax.broadcasted_iota(jnp.int32, v.shape, 0)
    targ = jnp.min(jnp.where(v == tmin, iota, bn), axis=0, keepdims=True) + j * bn
    better = tmin < min_ref[...]
    arg_ref[...] = jnp.where(better, targ, arg_ref[...])
    min_ref[...] = jnp.where(better, tmin, min_ref[...])

    @pl.when(j == nt - 1)
    def _fin():
        xs = x_ref[...]
        ones = jnp.ones((1, xs.shape[1]), jnp.bfloat16)
        x2 = jax.lax.dot_general(
            ones, xs * xs, (((1,), (1,)), ((), ())),
            preferred_element_type=jnp.float32) * 0.25
        mind = min_ref[...] + x2
        o_ref[...] = jnp.where(mind <= _THR, arg_ref[...], _NOC).astype(jnp.int32)


def _perm_matrix(p, C):
    D = p * p * C
    P = np.zeros((D, D), np.float32)
    for c in range(C):
        for q in range(p * p):
            P[c * p * p + q, q * C + c] = -2.0
    return jnp.asarray(P, dtype=jnp.bfloat16)


def kernel(imgs, patch_size, codes, active):
    B, C, T, H, W = imgs.shape
    N, D = codes.shape
    p = int(np.sqrt(D // C))
    Hp, Wp = H // p, W // p
    M = B * T * Hp * Wp
    R = Hp * Wp

    img6 = imgs.reshape(B, C, T, Hp, p, Wp * p)
    P = _perm_matrix(p, C)

    xs = pl.pallas_call(
        functools.partial(_patchify_kernel, C=C, Hp=Hp, Wp=Wp, p=p),
        grid=(B * T,),
        in_specs=[
            pl.BlockSpec((1, C, 1, Hp, p, Wp * p),
                         lambda k: (k // T, 0, k % T, 0, 0, 0)),
            pl.BlockSpec((D, D), lambda k: (0, 0)),
        ],
        out_specs=pl.BlockSpec((R, D), lambda k: (k, 0)),
        out_shape=jax.ShapeDtypeStruct((M, D), jnp.bfloat16),
    )(img6, P)

    BN = 1024
    NT = N // BN
    amask = active.astype(jnp.float32).reshape(N, 1)

    out = pl.pallas_call(
        functools.partial(_nn_kernel, nt=NT, bn=BN),
        grid=(NT,),
        in_specs=[
            pl.BlockSpec((M, D), lambda j: (0, 0)),
            pl.BlockSpec((BN, D), lambda j: (j, 0)),
            pl.BlockSpec((BN, 1), lambda j: (j, 0)),
        ],
        out_specs=pl.BlockSpec((1, M), lambda j: (0, 0)),
        out_shape=jax.ShapeDtypeStruct((1, M), jnp.int32),
        scratch_shapes=[
            pltpu.VMEM((1, M), jnp.float32),
            pltpu.VMEM((1, M), jnp.int32),
        ],
    )(xs, codes, amask)
    return out.reshape(B, T, Hp, Wp)
